# Initial kernel scaffold; baseline (speedup 1.0000x reference)
import jax
import jax.numpy as jnp
from jax.experimental import pallas as pl


def kernel(x, edge_index, batch, W1, b1, g1, be1, W2, b2, g2, be2, Wl1, bl1, Wl2, bl2):
    raise NotImplementedError("write your pallas kernel here")


# speedup vs baseline: 9.8387x; 9.8387x over previous
import functools

import jax
import jax.numpy as jnp
from jax import lax
from jax.experimental import pallas as pl
from jax.experimental.pallas import tpu as pltpu
from jax.experimental.pallas import tpu_sc as plsc

N = 10000
E = 160000
FIN = 256
HID = 512
NGRAPH = 64

NC, NS = 2, 16
NTILE = NC * NS

EHG = 40
EH = EHG * 128
HSIZE = 10240
HSLC = HSIZE // NS

EGRP = 128
NGRP = 79
EPT = EGRP * NGRP
EPAD = NS * EPT
ACC_N = 10112
SLC = ACC_N // NS

RB = 400
GRID = N // RB

_f32 = jnp.float32


@functools.partial(
    pl.kernel,
    out_type=jax.ShapeDtypeStruct((NC * HSIZE,), _f32),
    mesh=plsc.VectorSubcoreMesh(core_axis_name="c", subcore_axis_name="s",
                                num_cores=NC, num_subcores=NS),
    scratch_types=[
        pltpu.VMEM((EHG, 128), jnp.int32),
        pltpu.VMEM((128,), _f32),
        pltpu.VMEM_SHARED((HSIZE,), _f32),
    ],
)
def _sc_hist(colh_hbm, ones_hbm, zeros_hbm, out_hbm, colv, onesv, hist):
    cid = lax.axis_index("c")
    sid = lax.axis_index("s")
    wid = cid * NS + sid
    pltpu.sync_copy(colh_hbm.at[wid], colv)
    pltpu.sync_copy(ones_hbm, onesv)
    pltpu.sync_copy(zeros_hbm.at[pl.ds(sid * HSLC, HSLC)],
                    hist.at[pl.ds(sid * HSLC, HSLC)])
    plsc.subcore_barrier()

    def hb(g, carry):
        pltpu.sync_copy(onesv, hist.at[colv.at[g]], add=True)
        return carry

    lax.fori_loop(0, EHG, hb, 0)
    plsc.subcore_barrier()
    pltpu.sync_copy(hist.at[pl.ds(sid * HSLC, HSLC)],
                    out_hbm.at[pl.ds(cid * HSIZE + sid * HSLC, HSLC)])


def _make_agg(nch):
    cpc = nch // NC

    @functools.partial(
        pl.kernel,
        out_type=jax.ShapeDtypeStruct((nch, ACC_N, 128), _f32),
        mesh=plsc.VectorSubcoreMesh(core_axis_name="c", subcore_axis_name="s",
                                    num_cores=NC, num_subcores=NS),
        scratch_types=[
            pltpu.VMEM((NGRP, EGRP), jnp.int32),
            pltpu.VMEM((NGRP, EGRP), jnp.int32),
            pltpu.VMEM((EGRP, 128), _f32),
            pltpu.VMEM_SHARED((ACC_N, 128), _f32),
            pltpu.SemaphoreType.DMA,
        ],
    )
    def agg(xs_hbm, rows_hbm, cols_hbm, zeros_hbm, out_hbm,
            rowv, colv, buf, accum, sem):
        cid = lax.axis_index("c")
        sid = lax.axis_index("s")
        pltpu.sync_copy(rows_hbm.at[sid], rowv)
        pltpu.sync_copy(cols_hbm.at[sid], colv)
        for ch in range(cpc):
            chunk = cid * cpc + ch
            pltpu.sync_copy(zeros_hbm, accum.at[pl.ds(sid * SLC, SLC)])
            plsc.subcore_barrier()

            def gb(g, carry):
                pltpu.async_copy(
                    xs_hbm.at[chunk].at[rowv.at[g]], buf, sem).wait()
                pltpu.sync_copy(buf, accum.at[colv.at[g]], add=True)
                return carry

            lax.fori_loop(0, NGRP, gb, 0)
            plsc.subcore_barrier()
            pltpu.sync_copy(accum.at[pl.ds(sid * SLC, SLC)],
                            out_hbm.at[chunk].at[pl.ds(sid * SLC, SLC)])

    return agg


_agg2 = _make_agg(2)
_agg4 = _make_agg(4)


def _dinv_body(p_ref, out_ref):
    deg = lax.dot_general(p_ref[...], jnp.ones((NC, 1), _f32),
                          (((0,), (0,)), ((), ())),
                          preferred_element_type=_f32) + 1.0
    out_ref[...] = jnp.broadcast_to(jax.lax.rsqrt(deg), (HSIZE, 8))


_dinv_call = pl.pallas_call(
    _dinv_body,
    out_shape=jax.ShapeDtypeStruct((HSIZE, 8), _f32),
)


def _scale_x_body(x_ref, d_ref, out_ref):
    xs = x_ref[...] * d_ref[:, :1]
    out_ref[...] = jnp.stack([xs[:, :128], xs[:, 128:]], axis=0)


_scale_x_call = pl.pallas_call(
    _scale_x_body,
    grid=(GRID,),
    in_specs=[
        pl.BlockSpec((RB, FIN), lambda i: (i, 0)),
        pl.BlockSpec((RB, 8), lambda i: (i, 0)),
    ],
    out_specs=pl.BlockSpec((2, RB, 128), lambda i: (0, i, 0)),
    out_shape=jax.ShapeDtypeStruct((2, N, 128), _f32),
)


def _l1_body(agg_ref, xs_ref, d_ref, w1_ref, b1_ref,
             pre_ref, mean_ref, var_ref, ssum, ssq):
    i = pl.program_id(0)
    a = jnp.concatenate([agg_ref[0], agg_ref[1]], axis=1)
    xsv = jnp.concatenate([xs_ref[0], xs_ref[1]], axis=1)
    n1 = d_ref[:, :1] * (a + xsv)
    pre = jnp.dot(n1, w1_ref[...], preferred_element_type=_f32) + b1_ref[...]
    pre_ref[...] = pre
    part = jnp.sum(pre.reshape(RB // 8, 8, HID), axis=0)
    partsq = jnp.sum((pre * pre).reshape(RB // 8, 8, HID), axis=0)

    @pl.when(i == 0)
    def _():
        ssum[...] = part
        ssq[...] = partsq

    @pl.when(i > 0)
    def _():
        ssum[...] += part
        ssq[...] += partsq

    @pl.when(i == GRID - 1)
    def _():
        tot = jnp.sum(ssum[...], axis=0, keepdims=True) / N
        totsq = jnp.sum(ssq[...], axis=0, keepdims=True) / N
        mean_ref[...] = jnp.broadcast_to(tot, (8, HID))
        var_ref[...] = jnp.broadcast_to(totsq - tot * tot, (8, HID))


_l1_call = pl.pallas_call(
    _l1_body,
    grid=(GRID,),
    in_specs=[
        pl.BlockSpec((2, RB, 128), lambda i: (0, i, 0)),
        pl.BlockSpec((2, RB, 128), lambda i: (0, i, 0)),
        pl.BlockSpec((RB, 8), lambda i: (i, 0)),
        pl.BlockSpec((FIN, HID), lambda i: (0, 0)),
        pl.BlockSpec((1, HID), lambda i: (0, 0)),
    ],
    out_specs=[
        pl.BlockSpec((RB, HID), lambda i: (i, 0)),
        pl.BlockSpec((8, HID), lambda i: (0, 0)),
        pl.BlockSpec((8, HID), lambda i: (0, 0)),
    ],
    out_shape=[
        jax.ShapeDtypeStruct((N, HID), _f32),
        jax.ShapeDtypeStruct((8, HID), _f32),
        jax.ShapeDtypeStruct((8, HID), _f32),
    ],
    scratch_shapes=[pltpu.VMEM((8, HID), _f32), pltpu.VMEM((8, HID), _f32)],
)


def _l2in_body(pre_ref, mean_ref, var_ref, g_ref, be_ref, d_ref, w2_ref,
               out_ref):
    m = mean_ref[0:1, :]
    v = var_ref[0:1, :]
    h = (pre_ref[...] - m) * jax.lax.rsqrt(v + 1e-5) * g_ref[...] + be_ref[...]
    h = jnp.maximum(h, 0.0)
    xw2 = jnp.dot(h, w2_ref[...], preferred_element_type=_f32)
    xs2 = d_ref[:, :1] * xw2
    out_ref[...] = jnp.stack(
        [xs2[:, 0:128], xs2[:, 128:256], xs2[:, 256:384], xs2[:, 384:512]],
        axis=0)


_l2in_call = pl.pallas_call(
    _l2in_body,
    grid=(GRID,),
    in_specs=[
        pl.BlockSpec((RB, HID), lambda i: (i, 0)),
        pl.BlockSpec((8, HID), lambda i: (0, 0)),
        pl.BlockSpec((8, HID), lambda i: (0, 0)),
        pl.BlockSpec((1, HID), lambda i: (0, 0)),
        pl.BlockSpec((1, HID), lambda i: (0, 0)),
        pl.BlockSpec((RB, 8), lambda i: (i, 0)),
        pl.BlockSpec((HID, HID), lambda i: (0, 0)),
    ],
    out_specs=pl.BlockSpec((4, RB, 128), lambda i: (0, i, 0)),
    out_shape=jax.ShapeDtypeStruct((4, N, 128), _f32),
)


def _l2out_body(agg_ref, xs_ref, d_ref, b2_ref,
                pre_ref, mean_ref, var_ref, ssum, ssq):
    i = pl.program_id(0)
    a = jnp.concatenate([agg_ref[0], agg_ref[1], agg_ref[2], agg_ref[3]],
                        axis=1)
    xsv = jnp.concatenate([xs_ref[0], xs_ref[1], xs_ref[2], xs_ref[3]],
                          axis=1)
    pre = d_ref[:, :1] * (a + xsv) + b2_ref[...]
    pre_ref[...] = pre
    part = jnp.sum(pre.reshape(RB // 8, 8, HID), axis=0)
    partsq = jnp.sum((pre * pre).reshape(RB // 8, 8, HID), axis=0)

    @pl.when(i == 0)
    def _():
        ssum[...] = part
        ssq[...] = partsq

    @pl.when(i > 0)
    def _():
        ssum[...] += part
        ssq[...] += partsq

    @pl.when(i == GRID - 1)
    def _():
        tot = jnp.sum(ssum[...], axis=0, keepdims=True) / N
        totsq = jnp.sum(ssq[...], axis=0, keepdims=True) / N
        mean_ref[...] = jnp.broadcast_to(tot, (8, HID))
        var_ref[...] = jnp.broadcast_to(totsq - tot * tot, (8, HID))


_l2out_call = pl.pallas_call(
    _l2out_body,
    grid=(GRID,),
    in_specs=[
        pl.BlockSpec((4, RB, 128), lambda i: (0, i, 0)),
        pl.BlockSpec((4, RB, 128), lambda i: (0, i, 0)),
        pl.BlockSpec((RB, 8), lambda i: (i, 0)),
        pl.BlockSpec((1, HID), lambda i: (0, 0)),
    ],
    out_specs=[
        pl.BlockSpec((RB, HID), lambda i: (i, 0)),
        pl.BlockSpec((8, HID), lambda i: (0, 0)),
        pl.BlockSpec((8, HID), lambda i: (0, 0)),
    ],
    out_shape=[
        jax.ShapeDtypeStruct((N, HID), _f32),
        jax.ShapeDtypeStruct((8, HID), _f32),
        jax.ShapeDtypeStruct((8, HID), _f32),
    ],
    scratch_shapes=[pltpu.VMEM((8, HID), _f32), pltpu.VMEM((8, HID), _f32)],
)


def _head_body(pre_ref, mean_ref, var_ref, g_ref, be_ref, batch_ref,
               wl1_ref, bl1_ref, wl2_ref, bl2_ref, out_ref, psum, pcnt):
    i = pl.program_id(0)
    m = mean_ref[0:1, :]
    v = var_ref[0:1, :]
    h = (pre_ref[...] - m) * jax.lax.rsqrt(v + 1e-5) * g_ref[...] + be_ref[...]
    h = jnp.maximum(h, 0.0)
    b = batch_ref[0, 0, :]
    gid = lax.broadcasted_iota(jnp.int32, (NGRAPH, RB), 0)
    sel = (b[None, :] == gid).astype(_f32)
    ps = jnp.dot(sel, h, preferred_element_type=_f32)
    cs = jnp.sum(sel, axis=1, keepdims=True)

    @pl.when(i == 0)
    def _():
        psum[...] = ps
        pcnt[...] = jnp.broadcast_to(cs, (NGRAPH, 128))

    @pl.when(i > 0)
    def _():
        psum[...] += ps
        pcnt[...] += jnp.broadcast_to(cs, (NGRAPH, 128))

    @pl.when(i == GRID - 1)
    def _():
        cnt = jnp.maximum(pcnt[:, :1], 1.0)
        pooled = psum[...] / cnt
        z = jnp.dot(pooled, wl1_ref[...], preferred_element_type=_f32)
        z = jnp.maximum(z + bl1_ref[...], 0.0)
        o = jnp.dot(z, wl2_ref[...], preferred_element_type=_f32) + bl2_ref[...]
        out_ref[...] = jnp.broadcast_to(o, (NGRAPH, 8))


_head_call = pl.pallas_call(
    _head_body,
    grid=(GRID,),
    in_specs=[
        pl.BlockSpec((RB, HID), lambda i: (i, 0)),
        pl.BlockSpec((8, HID), lambda i: (0, 0)),
        pl.BlockSpec((8, HID), lambda i: (0, 0)),
        pl.BlockSpec((1, HID), lambda i: (0, 0)),
        pl.BlockSpec((1, HID), lambda i: (0, 0)),
        pl.BlockSpec((1, 1, RB), lambda i: (i, 0, 0)),
        pl.BlockSpec((HID, HID // 2), lambda i: (0, 0)),
        pl.BlockSpec((1, HID // 2), lambda i: (0, 0)),
        pl.BlockSpec((HID // 2, 1), lambda i: (0, 0)),
        pl.BlockSpec((1, 1), lambda i: (0, 0)),
    ],
    out_specs=pl.BlockSpec((NGRAPH, 8), lambda i: (0, 0)),
    out_shape=jax.ShapeDtypeStruct((NGRAPH, 8), _f32),
    scratch_shapes=[pltpu.VMEM((NGRAPH, HID), _f32),
                    pltpu.VMEM((NGRAPH, 128), _f32)],
)


def kernel(x, edge_index, batch, W1, b1, g1, be1, W2, b2, g2, be2,
           Wl1, bl1, Wl2, bl2):
    ei = edge_index.astype(jnp.int32)
    row, col = ei[0], ei[1]

    colh = jnp.full((NTILE * EH,), N, jnp.int32).at[:E].set(col)
    colh = colh.reshape(NTILE, EHG, 128)
    rowp = jnp.zeros((EPAD,), jnp.int32).at[:E].set(row).reshape(
        NS, NGRP, EGRP)
    colp = jnp.full((EPAD,), N, jnp.int32).at[:E].set(col).reshape(
        NS, NGRP, EGRP)
    zrows = jnp.zeros((SLC, 128), _f32)
    batch3 = batch.astype(jnp.int32).reshape(GRID, 1, RB)

    partials = _sc_hist(colh, jnp.ones((128,), _f32),
                        jnp.zeros((HSIZE,), _f32))
    dinv = _dinv_call(partials.reshape(NC, HSIZE))

    xs1 = _scale_x_call(x, dinv)
    agg1 = _agg2(xs1, rowp, colp, zrows)
    pre1, mean1, var1 = _l1_call(agg1, xs1, dinv, W1, b1.reshape(1, HID))

    xs2 = _l2in_call(pre1, mean1, var1, g1.reshape(1, HID),
                     be1.reshape(1, HID), dinv, W2)
    agg2 = _agg4(xs2, rowp, colp, zrows)
    pre2, mean2, var2 = _l2out_call(agg2, xs2, dinv, b2.reshape(1, HID))

    out = _head_call(pre2, mean2, var2, g2.reshape(1, HID),
                     be2.reshape(1, HID), batch3,
                     Wl1, bl1.reshape(1, HID // 2),
                     Wl2, bl2.reshape(1, 1))
    return out[:, 0]
